# baseline (speedup 1.0000x reference)
import functools

import jax
import jax.numpy as jnp
from jax import lax
from jax.experimental import pallas as pl
from jax.experimental.pallas import tpu as pltpu
from jax.experimental.pallas import tpu_sc as plsc

N = 10000
E = 320000
D = 128
HD = 64
DG = 16
NC = 2
NS = 16
C = 80
NCHUNK = E // (NS * C)
ROWS_PER_TILE = N // NS
OROWS = 125
QD = 32
NPIECE = (NCHUNK + QD - 1) // QD
NBUF = 3

_mesh = plsc.VectorSubcoreMesh(
    core_axis_name="c", subcore_axis_name="s", num_cores=NC, num_subcores=NS
)


def _piece_len(p):
    return min(QD, NCHUNK - p * QD)


@functools.partial(
    pl.kernel,
    out_type=(
        jax.ShapeDtypeStruct((N, D), jnp.float32),
        jax.ShapeDtypeStruct((NC, N, DG), jnp.float32),
    ),
    mesh=_mesh,
    scratch_types=[
        pltpu.VMEM((2, QD, C), jnp.int32),
        pltpu.VMEM((QD, C), jnp.int32),
        pltpu.VMEM((NBUF, C, HD), jnp.float32),
        pltpu.VMEM((C, DG), jnp.float32),
        pltpu.VMEM_SHARED((N, HD), jnp.float32),
        pltpu.VMEM_SHARED((N, HD), jnp.float32),
        pltpu.VMEM_SHARED((N, DG), jnp.float32),
        pltpu.SemaphoreType.DMA((NBUF,)),
        pltpu.SemaphoreType.DMA((NBUF,)),
        pltpu.SemaphoreType.DMA((NBUF,)),
    ],
    compiler_params=pltpu.CompilerParams(use_tc_tiling_on_sc=False),
)
def _sc_aggregate(x2_hbm, edge_hbm, z64_hbm, z16_hbm, ones_hbm,
                  out_hbm, deg_hbm,
                  src_v, dst_v, rows_v, ones_v, xs, acc, dacc,
                  sem, sem_s, sem_d):
    cid = lax.axis_index("c")
    sid = lax.axis_index("s")
    src_hbm = edge_hbm.at[0]
    dst_hbm = edge_hbm.at[1]

    for kz in range(ROWS_PER_TILE // OROWS):
        r0 = sid * ROWS_PER_TILE + kz * OROWS
        pltpu.async_copy(z64_hbm, acc.at[pl.ds(r0, OROWS)], sem_s.at[0])
        pltpu.async_copy(z16_hbm, dacc.at[pl.ds(r0, OROWS)], sem_s.at[1])
    pltpu.async_copy(x2_hbm.at[cid].at[pl.ds(sid * ROWS_PER_TILE, ROWS_PER_TILE)],
                     xs.at[pl.ds(sid * ROWS_PER_TILE, ROWS_PER_TILE)], sem_s.at[2])
    pltpu.async_copy(ones_hbm, ones_v, sem_d.at[0])

    pltpu.sync_copy(src_hbm.at[sid].at[pl.ds(0, QD)], src_v.at[0])
    pltpu.sync_copy(src_hbm.at[sid].at[pl.ds(QD, QD)], src_v.at[1])
    pltpu.sync_copy(dst_hbm.at[sid].at[pl.ds(0, QD)], dst_v)

    for kz in range(ROWS_PER_TILE // OROWS):
        r0 = sid * ROWS_PER_TILE + kz * OROWS
        pltpu.make_async_copy(z64_hbm, acc.at[pl.ds(r0, OROWS)], sem_s.at[0]).wait()
        pltpu.make_async_copy(z16_hbm, dacc.at[pl.ds(r0, OROWS)], sem_s.at[1]).wait()
    pltpu.make_async_copy(x2_hbm.at[cid].at[pl.ds(0, ROWS_PER_TILE)],
                          xs.at[pl.ds(0, ROWS_PER_TILE)], sem_s.at[2]).wait()
    pltpu.make_async_copy(ones_hbm, ones_v, sem_d.at[0]).wait()
    plsc.subcore_barrier()

    def _gather(j, buf):
        slot = lax.rem(j // QD, 2)
        row = lax.rem(j, QD)
        pltpu.async_copy(xs.at[src_v.at[slot].at[row]], rows_v.at[buf],
                         sem.at[buf])

    def _gather_wait(buf):
        pltpu.make_async_copy(
            xs.at[src_v.at[0].at[0]], rows_v.at[buf], sem.at[buf]).wait()

    def _scatter_wait(buf):
        pltpu.make_async_copy(
            rows_v.at[buf], acc.at[dst_v.at[0]], sem_s.at[buf]).wait()

    def _dscatter_wait(buf):
        pltpu.make_async_copy(
            ones_v, dacc.at[dst_v.at[0]], sem_d.at[buf]).wait()

    _gather(0, 0)
    _gather(1, 1)

    def _edge_chunk(j, _):
        buf = lax.rem(j, NBUF)
        _gather_wait(buf)

        @pl.when(j >= 1)
        def _():
            _scatter_wait(lax.rem(j - 1, NBUF))
            _dscatter_wait(lax.rem(j - 1, NBUF))

        for p in range(1, NPIECE):
            @pl.when(j == QD * p)
            def _(p=p):
                if p + 1 < NPIECE:
                    pltpu.sync_copy(
                        src_hbm.at[sid].at[pl.ds(QD * (p + 1), _piece_len(p + 1))],
                        src_v.at[(p + 1) % 2].at[pl.ds(0, _piece_len(p + 1))])
                pltpu.sync_copy(
                    dst_hbm.at[sid].at[pl.ds(QD * p, _piece_len(p))],
                    dst_v.at[pl.ds(0, _piece_len(p))])

        @pl.when(j + 2 < NCHUNK)
        def _():
            _gather(j + 2, lax.rem(j + 2, NBUF))

        drow = lax.rem(j, QD)
        pltpu.async_copy(rows_v.at[buf], acc.at[dst_v.at[drow]],
                         sem_s.at[buf], add=True)
        pltpu.async_copy(ones_v, dacc.at[dst_v.at[drow]],
                         sem_d.at[buf], add=True)
        return 0
    lax.fori_loop(0, NCHUNK, _edge_chunk, 0)
    _scatter_wait((NCHUNK - 1) % NBUF)
    _dscatter_wait((NCHUNK - 1) % NBUF)

    plsc.subcore_barrier()
    for kz in range(ROWS_PER_TILE // OROWS):
        r0 = sid * ROWS_PER_TILE + kz * OROWS
        pltpu.async_copy(acc.at[pl.ds(r0, OROWS)],
                         out_hbm.at[pl.ds(r0, OROWS), pl.ds(cid * HD, HD)],
                         sem.at[0])
    pltpu.async_copy(dacc.at[pl.ds(sid * ROWS_PER_TILE, ROWS_PER_TILE)],
                     deg_hbm.at[cid].at[pl.ds(sid * ROWS_PER_TILE, ROWS_PER_TILE)],
                     sem.at[1])
    for kz in range(ROWS_PER_TILE // OROWS):
        r0 = sid * ROWS_PER_TILE + kz * OROWS
        pltpu.make_async_copy(acc.at[pl.ds(r0, OROWS)],
                              out_hbm.at[pl.ds(r0, OROWS), pl.ds(cid * HD, HD)],
                              sem.at[0]).wait()
    pltpu.make_async_copy(dacc.at[pl.ds(sid * ROWS_PER_TILE, ROWS_PER_TILE)],
                          deg_hbm.at[cid].at[pl.ds(sid * ROWS_PER_TILE, ROWS_PER_TILE)],
                          sem.at[1]).wait()


def _tc_finish(p_ref, d_ref, x_ref, w_ref, b_ref, o_ref):
    deg = jnp.maximum(d_ref[0][:, 0:1], 1.0)
    h = p_ref[...] / deg + x_ref[...]
    y = jnp.dot(h, w_ref[...], preferred_element_type=jnp.float32) + b_ref[...]
    o_ref[...] = jnp.maximum(y, 0.0)


def kernel(x, edge_index, W, b):
    ei = edge_index.astype(jnp.int32).reshape(2, NS, NCHUNK, C)
    x2 = jnp.stack([x[:, :HD], x[:, HD:]])
    z64 = jnp.zeros((OROWS, HD), jnp.float32)
    z16 = jnp.zeros((OROWS, DG), jnp.float32)
    ones16 = jnp.ones((C, DG), jnp.float32)
    agg, degs = _sc_aggregate(x2, ei, z64, z16, ones16)

    BN = 1000
    out = pl.pallas_call(
        _tc_finish,
        grid=(N // BN,),
        in_specs=[
            pl.BlockSpec((BN, D), lambda i: (i, 0)),
            pl.BlockSpec((1, BN, DG), lambda i: (0, i, 0)),
            pl.BlockSpec((BN, D), lambda i: (i, 0)),
            pl.BlockSpec((D, D), lambda i: (0, 0)),
            pl.BlockSpec((1, D), lambda i: (0, 0)),
        ],
        out_specs=pl.BlockSpec((BN, D), lambda i: (i, 0)),
        out_shape=jax.ShapeDtypeStruct((N, D), jnp.float32),
    )(agg, degs, x, W, b.reshape(1, D))
    return out


# speedup vs baseline: 1.5925x; 1.5925x over previous
import functools

import jax
import jax.numpy as jnp
from jax import lax
from jax.experimental import pallas as pl
from jax.experimental.pallas import tpu as pltpu
from jax.experimental.pallas import tpu_sc as plsc

N = 10000
E = 320000
D = 128
DG = 16
NC = 2
NS = 16
NW = NC * NS
C = 80
NCHUNK = E // (NW * C)
ROWS_PER_TILE = N // NS
OROWS = 125
IH = 63
QD = 32
NBUF = 3

_mesh = plsc.VectorSubcoreMesh(
    core_axis_name="c", subcore_axis_name="s", num_cores=NC, num_subcores=NS
)


@functools.partial(
    pl.kernel,
    out_type=(
        jax.ShapeDtypeStruct((NC, N, D), jnp.float32),
        jax.ShapeDtypeStruct((NC, N, DG), jnp.float32),
    ),
    mesh=_mesh,
    scratch_types=[
        pltpu.VMEM((IH, C), jnp.int32),
        pltpu.VMEM((QD, C), jnp.int32),
        pltpu.VMEM((NBUF, C, D), jnp.float32),
        pltpu.VMEM((C, DG), jnp.float32),
        pltpu.VMEM_SHARED((N, D), jnp.float32),
        pltpu.VMEM_SHARED((N, DG), jnp.float32),
        pltpu.SemaphoreType.DMA((NBUF,)),
        pltpu.SemaphoreType.DMA((NBUF,)),
        pltpu.SemaphoreType.DMA((NBUF,)),
    ],
    compiler_params=pltpu.CompilerParams(use_tc_tiling_on_sc=False),
)
def _sc_aggregate(x_hbm, edge_hbm, z128_hbm, z16_hbm, ones_hbm,
                  out_hbm, deg_hbm,
                  src_v, dst_v, rows_v, ones_v, acc, dacc,
                  sem, sem_s, sem_d):
    cid = lax.axis_index("c")
    sid = lax.axis_index("s")
    wid = cid * NS + sid
    src_hbm = edge_hbm.at[0]
    dst_hbm = edge_hbm.at[1]

    for kz in range(ROWS_PER_TILE // OROWS):
        r0 = sid * ROWS_PER_TILE + kz * OROWS
        pltpu.async_copy(z128_hbm, acc.at[pl.ds(r0, OROWS)], sem_s.at[0])
        pltpu.async_copy(z16_hbm, dacc.at[pl.ds(r0, OROWS)], sem_s.at[1])
    pltpu.async_copy(ones_hbm, ones_v, sem_s.at[2])

    pltpu.sync_copy(src_hbm.at[wid].at[pl.ds(0, IH)], src_v)
    pltpu.sync_copy(dst_hbm.at[wid].at[pl.ds(0, QD)], dst_v)

    def _gather(j, buf):
        row = jnp.where(j < IH, j, j - IH)
        pltpu.async_copy(x_hbm.at[src_v.at[row].at[pl.ds(0, C // 2)]],
                         rows_v.at[buf].at[pl.ds(0, C // 2)], sem.at[buf])
        pltpu.async_copy(x_hbm.at[src_v.at[row].at[pl.ds(C // 2, C // 2)]],
                         rows_v.at[buf].at[pl.ds(C // 2, C // 2)], sem.at[buf])

    def _gather_wait(buf):
        pltpu.make_async_copy(
            x_hbm.at[src_v.at[0]], rows_v.at[buf], sem.at[buf]).wait()

    def _scatter_wait(buf):
        pltpu.make_async_copy(
            rows_v.at[buf], acc.at[dst_v.at[0]], sem_s.at[buf]).wait()

    def _dscatter_wait(buf):
        pltpu.make_async_copy(
            ones_v, dacc.at[dst_v.at[0]], sem_d.at[buf]).wait()

    _gather(0, 0)
    _gather(1, 1)
    for kz in range(ROWS_PER_TILE // OROWS):
        r0 = sid * ROWS_PER_TILE + kz * OROWS
        pltpu.make_async_copy(z128_hbm, acc.at[pl.ds(r0, OROWS)], sem_s.at[0]).wait()
        pltpu.make_async_copy(z16_hbm, dacc.at[pl.ds(r0, OROWS)], sem_s.at[1]).wait()
    pltpu.make_async_copy(ones_hbm, ones_v, sem_s.at[2]).wait()
    plsc.subcore_barrier()


    def _edge_chunk(j, _):
        buf = lax.rem(j, NBUF)
        _gather_wait(buf)

        @pl.when(j >= 1)
        def _():
            _scatter_wait(lax.rem(j - 1, NBUF))
            _dscatter_wait(lax.rem(j - 1, NBUF))

        @pl.when(j == IH - 2)
        def _():
            pltpu.sync_copy(src_hbm.at[wid].at[pl.ds(IH, NCHUNK - IH)],
                            src_v.at[pl.ds(0, NCHUNK - IH)])

        for jq, qn in ((QD, QD), (2 * QD, QD), (3 * QD, NCHUNK - 3 * QD)):
            @pl.when(j == jq)
            def _(jq=jq, qn=qn):
                pltpu.sync_copy(dst_hbm.at[wid].at[pl.ds(jq, qn)],
                                dst_v.at[pl.ds(0, qn)])

        @pl.when(j + 2 < NCHUNK)
        def _():
            _gather(j + 2, lax.rem(j + 2, NBUF))

        drow = lax.rem(j, QD)
        pltpu.async_copy(rows_v.at[buf], acc.at[dst_v.at[drow]],
                         sem_s.at[buf], add=True)
        pltpu.async_copy(ones_v, dacc.at[dst_v.at[drow]],
                         sem_d.at[buf], add=True)
        return 0
    lax.fori_loop(0, NCHUNK, _edge_chunk, 0)
    _scatter_wait((NCHUNK - 1) % NBUF)
    _dscatter_wait((NCHUNK - 1) % NBUF)

    plsc.subcore_barrier()
    for kz in range(ROWS_PER_TILE // OROWS):
        r0 = sid * ROWS_PER_TILE + kz * OROWS
        pltpu.async_copy(acc.at[pl.ds(r0, OROWS)],
                         out_hbm.at[cid].at[pl.ds(r0, OROWS)], sem.at[0])
    pltpu.async_copy(dacc.at[pl.ds(sid * ROWS_PER_TILE, ROWS_PER_TILE)],
                     deg_hbm.at[cid].at[pl.ds(sid * ROWS_PER_TILE, ROWS_PER_TILE)],
                     sem.at[1])
    for kz in range(ROWS_PER_TILE // OROWS):
        r0 = sid * ROWS_PER_TILE + kz * OROWS
        pltpu.make_async_copy(acc.at[pl.ds(r0, OROWS)],
                              out_hbm.at[cid].at[pl.ds(r0, OROWS)], sem.at[0]).wait()
    pltpu.make_async_copy(dacc.at[pl.ds(sid * ROWS_PER_TILE, ROWS_PER_TILE)],
                          deg_hbm.at[cid].at[pl.ds(sid * ROWS_PER_TILE, ROWS_PER_TILE)],
                          sem.at[1]).wait()


def _tc_finish(p_ref, d_ref, x_ref, w_ref, b_ref, o_ref):
    agg = p_ref[0] + p_ref[1]
    deg16 = d_ref[0] + d_ref[1]
    deg = jnp.maximum(deg16[:, 0:1], 1.0)
    h = agg / deg + x_ref[...]
    y = jnp.dot(h, w_ref[...], preferred_element_type=jnp.float32) + b_ref[...]
    o_ref[...] = jnp.maximum(y, 0.0)


def kernel(x, edge_index, W, b):
    ei = edge_index.astype(jnp.int32).reshape(2, NW, NCHUNK, C)
    z128 = jnp.zeros((OROWS, D), jnp.float32)
    z16 = jnp.zeros((OROWS, DG), jnp.float32)
    ones16 = jnp.ones((C, DG), jnp.float32)
    partials, degs = _sc_aggregate(x, ei, z128, z16, ones16)

    BN = 1000
    out = pl.pallas_call(
        _tc_finish,
        grid=(N // BN,),
        in_specs=[
            pl.BlockSpec((NC, BN, D), lambda i: (0, i, 0)),
            pl.BlockSpec((NC, BN, DG), lambda i: (0, i, 0)),
            pl.BlockSpec((BN, D), lambda i: (i, 0)),
            pl.BlockSpec((D, D), lambda i: (0, 0)),
            pl.BlockSpec((1, D), lambda i: (0, 0)),
        ],
        out_specs=pl.BlockSpec((BN, D), lambda i: (i, 0)),
        out_shape=jax.ShapeDtypeStruct((N, D), jnp.float32),
    )(partials, degs, x, W, b.reshape(1, D))
    return out


# speedup vs baseline: 1.6401x; 1.0299x over previous
import functools

import jax
import jax.numpy as jnp
from jax import lax
from jax.experimental import pallas as pl
from jax.experimental.pallas import tpu as pltpu
from jax.experimental.pallas import tpu_sc as plsc

N = 10000
E = 320000
D = 128
DG = 16
NC = 2
NS = 16
NW = NC * NS
C = 80
NCHUNK = E // (NW * C)
ROWS_PER_TILE = N // NS
OROWS = 125
IH = 63
QD = 32
NBUF = 3

_mesh = plsc.VectorSubcoreMesh(
    core_axis_name="c", subcore_axis_name="s", num_cores=NC, num_subcores=NS
)


@functools.partial(
    pl.kernel,
    out_type=(
        jax.ShapeDtypeStruct((NC, N, D), jnp.float32),
        jax.ShapeDtypeStruct((NC, N, DG), jnp.float32),
    ),
    mesh=_mesh,
    scratch_types=[
        pltpu.VMEM((IH, C), jnp.int32),
        pltpu.VMEM((QD, C), jnp.int32),
        pltpu.VMEM((NBUF, C, D), jnp.float32),
        pltpu.VMEM((C, DG), jnp.float32),
        pltpu.VMEM_SHARED((N, D), jnp.float32),
        pltpu.VMEM_SHARED((N, DG), jnp.float32),
        pltpu.SemaphoreType.DMA((NBUF,)),
        pltpu.SemaphoreType.DMA((NBUF,)),
        pltpu.SemaphoreType.DMA((NBUF,)),
    ],
    compiler_params=pltpu.CompilerParams(use_tc_tiling_on_sc=False),
)
def _sc_aggregate(x_hbm, edge_hbm, ones_hbm,
                  out_hbm, deg_hbm,
                  src_v, dst_v, rows_v, ones_v, acc, dacc,
                  sem, sem_s, sem_d):
    cid = lax.axis_index("c")
    sid = lax.axis_index("s")
    wid = cid * NS + sid
    src_hbm = edge_hbm.at[0]
    dst_hbm = edge_hbm.at[1]

    def _zrow(i, _):
        r = i // (D // 16)
        f = i % (D // 16)
        rows_v[2, r, pl.ds(f * 16, 16)] = jnp.zeros((16,), jnp.float32)
        return 0
    lax.fori_loop(0, C * (D // 16), _zrow, 0)
    zpage = rows_v.at[2]
    zcopies = []
    for kz in range(7):
        zcopies.append((zpage, acc.at[pl.ds(sid * ROWS_PER_TILE + kz * C, C)]))
    zcopies.append((zpage.at[pl.ds(0, 65)],
                    acc.at[pl.ds(sid * ROWS_PER_TILE + 7 * C, 65)]))
    for kz in range(7):
        zcopies.append((zpage.at[:, pl.ds(0, DG)],
                        dacc.at[pl.ds(sid * ROWS_PER_TILE + kz * C, C)]))
    zcopies.append((zpage.at[pl.ds(0, 65)].at[:, pl.ds(0, DG)],
                    dacc.at[pl.ds(sid * ROWS_PER_TILE + 7 * C, 65)]))
    for s_, d_ in zcopies:
        pltpu.async_copy(s_, d_, sem_s.at[0])
    pltpu.async_copy(ones_hbm, ones_v, sem_s.at[2])

    pltpu.sync_copy(src_hbm.at[wid].at[pl.ds(0, IH)], src_v)
    pltpu.sync_copy(dst_hbm.at[wid].at[pl.ds(0, QD)], dst_v)

    def _gather(j, buf):
        row = jnp.where(j < IH, j, j - IH)
        pltpu.async_copy(x_hbm.at[src_v.at[row].at[pl.ds(0, C // 2)]],
                         rows_v.at[buf].at[pl.ds(0, C // 2)], sem.at[buf])
        pltpu.async_copy(x_hbm.at[src_v.at[row].at[pl.ds(C // 2, C // 2)]],
                         rows_v.at[buf].at[pl.ds(C // 2, C // 2)], sem.at[buf])

    def _gather_wait(buf):
        pltpu.make_async_copy(
            x_hbm.at[src_v.at[0]], rows_v.at[buf], sem.at[buf]).wait()

    def _scatter_wait(buf):
        pltpu.make_async_copy(
            rows_v.at[buf], acc.at[dst_v.at[0]], sem_s.at[buf]).wait()

    def _dscatter_wait(buf):
        pltpu.make_async_copy(
            ones_v, dacc.at[dst_v.at[0]], sem_d.at[buf]).wait()

    _gather(0, 0)
    _gather(1, 1)
    for s_, d_ in zcopies:
        pltpu.make_async_copy(s_, d_, sem_s.at[0]).wait()
    pltpu.make_async_copy(ones_hbm, ones_v, sem_s.at[2]).wait()
    plsc.subcore_barrier()


    def _edge_chunk(j, _):
        buf = lax.rem(j, NBUF)
        _gather_wait(buf)

        @pl.when(j >= 1)
        def _():
            _scatter_wait(lax.rem(j - 1, NBUF))
            _dscatter_wait(lax.rem(j - 1, NBUF))

        @pl.when(j == IH - 2)
        def _():
            pltpu.sync_copy(src_hbm.at[wid].at[pl.ds(IH, NCHUNK - IH)],
                            src_v.at[pl.ds(0, NCHUNK - IH)])

        for jq, qn in ((QD, QD), (2 * QD, QD), (3 * QD, NCHUNK - 3 * QD)):
            @pl.when(j == jq)
            def _(jq=jq, qn=qn):
                pltpu.sync_copy(dst_hbm.at[wid].at[pl.ds(jq, qn)],
                                dst_v.at[pl.ds(0, qn)])

        @pl.when(j + 2 < NCHUNK)
        def _():
            _gather(j + 2, lax.rem(j + 2, NBUF))

        drow = lax.rem(j, QD)
        pltpu.async_copy(rows_v.at[buf], acc.at[dst_v.at[drow]],
                         sem_s.at[buf], add=True)
        pltpu.async_copy(ones_v, dacc.at[dst_v.at[drow]],
                         sem_d.at[buf], add=True)
        return 0
    lax.fori_loop(0, NCHUNK, _edge_chunk, 0)
    _scatter_wait((NCHUNK - 1) % NBUF)
    _dscatter_wait((NCHUNK - 1) % NBUF)

    plsc.subcore_barrier()
    for kz in range(ROWS_PER_TILE // OROWS):
        r0 = sid * ROWS_PER_TILE + kz * OROWS
        pltpu.async_copy(acc.at[pl.ds(r0, OROWS)],
                         out_hbm.at[cid].at[pl.ds(r0, OROWS)], sem.at[0])
    pltpu.async_copy(dacc.at[pl.ds(sid * ROWS_PER_TILE, ROWS_PER_TILE)],
                     deg_hbm.at[cid].at[pl.ds(sid * ROWS_PER_TILE, ROWS_PER_TILE)],
                     sem.at[1])
    for kz in range(ROWS_PER_TILE // OROWS):
        r0 = sid * ROWS_PER_TILE + kz * OROWS
        pltpu.make_async_copy(acc.at[pl.ds(r0, OROWS)],
                              out_hbm.at[cid].at[pl.ds(r0, OROWS)], sem.at[0]).wait()
    pltpu.make_async_copy(dacc.at[pl.ds(sid * ROWS_PER_TILE, ROWS_PER_TILE)],
                          deg_hbm.at[cid].at[pl.ds(sid * ROWS_PER_TILE, ROWS_PER_TILE)],
                          sem.at[1]).wait()


def _tc_finish(p_ref, d_ref, x_ref, w_ref, b_ref, o_ref):
    agg = p_ref[0] + p_ref[1]
    deg16 = d_ref[0] + d_ref[1]
    deg = jnp.maximum(deg16[:, 0:1], 1.0)
    h = agg / deg + x_ref[...]
    y = jnp.dot(h, w_ref[...], preferred_element_type=jnp.float32) + b_ref[...]
    o_ref[...] = jnp.maximum(y, 0.0)


def kernel(x, edge_index, W, b):
    ei = edge_index.astype(jnp.int32).reshape(2, NW, NCHUNK, C)
    ones16 = jnp.ones((C, DG), jnp.float32)
    partials, degs = _sc_aggregate(x, ei, ones16)

    BN = 1000
    out = pl.pallas_call(
        _tc_finish,
        grid=(N // BN,),
        in_specs=[
            pl.BlockSpec((NC, BN, D), lambda i: (0, i, 0)),
            pl.BlockSpec((NC, BN, DG), lambda i: (0, i, 0)),
            pl.BlockSpec((BN, D), lambda i: (i, 0)),
            pl.BlockSpec((D, D), lambda i: (0, 0)),
            pl.BlockSpec((1, D), lambda i: (0, 0)),
        ],
        out_specs=pl.BlockSpec((BN, D), lambda i: (i, 0)),
        out_shape=jax.ShapeDtypeStruct((N, D), jnp.float32),
    )(partials, degs, x, W, b.reshape(1, D))
    return out


# speedup vs baseline: 1.6619x; 1.0133x over previous
import functools

import jax
import jax.numpy as jnp
from jax import lax
from jax.experimental import pallas as pl
from jax.experimental.pallas import tpu as pltpu
from jax.experimental.pallas import tpu_sc as plsc

N = 10000
E = 320000
D = 128
DG = 16
NC = 2
NS = 16
NW = NC * NS
C = 80
NCHUNK = E // (NW * C)
ROWS_PER_TILE = N // NS
OROWS = 125
IH = 63
QD = 32
NBUF = 3

_mesh = plsc.VectorSubcoreMesh(
    core_axis_name="c", subcore_axis_name="s", num_cores=NC, num_subcores=NS
)


@functools.partial(
    pl.kernel,
    out_type=(
        jax.ShapeDtypeStruct((NC, N, D), jnp.float32),
        jax.ShapeDtypeStruct((NC, N, DG), jnp.float32),
    ),
    mesh=_mesh,
    scratch_types=[
        pltpu.VMEM((IH, C), jnp.int32),
        pltpu.VMEM((QD, C), jnp.int32),
        pltpu.VMEM((NBUF, C, D), jnp.float32),
        pltpu.VMEM((C, DG), jnp.float32),
        pltpu.VMEM_SHARED((N, D), jnp.float32),
        pltpu.VMEM_SHARED((N, DG), jnp.float32),
        pltpu.SemaphoreType.DMA((NBUF,)),
        pltpu.SemaphoreType.DMA((NBUF,)),
        pltpu.SemaphoreType.DMA((NBUF,)),
    ],
    compiler_params=pltpu.CompilerParams(use_tc_tiling_on_sc=False),
)
def _sc_aggregate(x_hbm, edge_hbm, ones_hbm,
                  out_hbm, deg_hbm,
                  src_v, dst_v, rows_v, ones_v, acc, dacc,
                  sem, sem_s, sem_d):
    cid = lax.axis_index("c")
    sid = lax.axis_index("s")
    wid = cid * NS + sid
    src_hbm = edge_hbm.at[0]
    dst_hbm = edge_hbm.at[1]

    def _zrow(i, _):
        r = i // (D // 16)
        f = i % (D // 16)
        rows_v[2, r, pl.ds(f * 16, 16)] = jnp.zeros((16,), jnp.float32)
        return 0
    lax.fori_loop(0, C * (D // 16), _zrow, 0)
    zpage = rows_v.at[2]
    zcopies = []
    for kz in range(7):
        zcopies.append((zpage, acc.at[pl.ds(sid * ROWS_PER_TILE + kz * C, C)]))
    zcopies.append((zpage.at[pl.ds(0, 65)],
                    acc.at[pl.ds(sid * ROWS_PER_TILE + 7 * C, 65)]))
    for kz in range(7):
        zcopies.append((zpage.at[:, pl.ds(0, DG)],
                        dacc.at[pl.ds(sid * ROWS_PER_TILE + kz * C, C)]))
    zcopies.append((zpage.at[pl.ds(0, 65)].at[:, pl.ds(0, DG)],
                    dacc.at[pl.ds(sid * ROWS_PER_TILE + 7 * C, 65)]))
    for s_, d_ in zcopies:
        pltpu.async_copy(s_, d_, sem_s.at[0])
    pltpu.async_copy(ones_hbm, ones_v, sem_s.at[2])

    pltpu.sync_copy(src_hbm.at[wid].at[pl.ds(0, IH)], src_v)
    pltpu.sync_copy(dst_hbm.at[wid].at[pl.ds(0, QD)], dst_v)

    def _gather(j, buf):
        row = jnp.where(j < IH, j, j - IH)
        pltpu.async_copy(x_hbm.at[src_v.at[row].at[pl.ds(0, C // 2)]],
                         rows_v.at[buf].at[pl.ds(0, C // 2)], sem.at[buf])
        pltpu.async_copy(x_hbm.at[src_v.at[row].at[pl.ds(C // 2, C // 2)]],
                         rows_v.at[buf].at[pl.ds(C // 2, C // 2)], sem.at[buf])

    def _gather_wait(buf):
        pltpu.make_async_copy(
            x_hbm.at[src_v.at[0]], rows_v.at[buf], sem.at[buf]).wait()

    def _scatter_wait(buf):
        pltpu.make_async_copy(
            rows_v.at[buf], acc.at[dst_v.at[0]], sem_s.at[buf]).wait()

    def _dscatter_wait(buf):
        pltpu.make_async_copy(
            ones_v, dacc.at[dst_v.at[0]], sem_d.at[buf]).wait()

    _gather(0, 0)
    _gather(1, 1)
    for s_, d_ in zcopies:
        pltpu.make_async_copy(s_, d_, sem_s.at[0]).wait()
    pltpu.make_async_copy(ones_hbm, ones_v, sem_s.at[2]).wait()
    plsc.subcore_barrier()


    def _edge_chunk(j, _):
        buf = lax.rem(j, NBUF)
        _gather_wait(buf)

        @pl.when(j >= 1)
        def _():
            _scatter_wait(lax.rem(j - 1, NBUF))
            _dscatter_wait(lax.rem(j - 1, NBUF))

        @pl.when(j == IH - 2)
        def _():
            pltpu.sync_copy(src_hbm.at[wid].at[pl.ds(IH, NCHUNK - IH)],
                            src_v.at[pl.ds(0, NCHUNK - IH)])

        for jq, qn in ((QD, QD), (2 * QD, QD), (3 * QD, NCHUNK - 3 * QD)):
            @pl.when(j == jq)
            def _(jq=jq, qn=qn):
                pltpu.sync_copy(dst_hbm.at[wid].at[pl.ds(jq, qn)],
                                dst_v.at[pl.ds(0, qn)])

        @pl.when(j + 2 < NCHUNK)
        def _():
            _gather(j + 2, lax.rem(j + 2, NBUF))

        drow = lax.rem(j, QD)
        pltpu.async_copy(rows_v.at[buf], acc.at[dst_v.at[drow]],
                         sem_s.at[buf], add=True)
        pltpu.async_copy(ones_v, dacc.at[dst_v.at[drow]],
                         sem_d.at[buf], add=True)
        return 0
    lax.fori_loop(0, NCHUNK, _edge_chunk, 0)
    _scatter_wait((NCHUNK - 1) % NBUF)
    _dscatter_wait((NCHUNK - 1) % NBUF)

    plsc.subcore_barrier()
    for kz in range(ROWS_PER_TILE // OROWS):
        r0 = sid * ROWS_PER_TILE + kz * OROWS
        pltpu.async_copy(acc.at[pl.ds(r0, OROWS)],
                         out_hbm.at[cid].at[pl.ds(r0, OROWS)], sem.at[0])
    pltpu.async_copy(dacc.at[pl.ds(sid * ROWS_PER_TILE, ROWS_PER_TILE)],
                     deg_hbm.at[cid].at[pl.ds(sid * ROWS_PER_TILE, ROWS_PER_TILE)],
                     sem.at[1])
    for kz in range(ROWS_PER_TILE // OROWS):
        r0 = sid * ROWS_PER_TILE + kz * OROWS
        pltpu.make_async_copy(acc.at[pl.ds(r0, OROWS)],
                              out_hbm.at[cid].at[pl.ds(r0, OROWS)], sem.at[0]).wait()
    pltpu.make_async_copy(dacc.at[pl.ds(sid * ROWS_PER_TILE, ROWS_PER_TILE)],
                          deg_hbm.at[cid].at[pl.ds(sid * ROWS_PER_TILE, ROWS_PER_TILE)],
                          sem.at[1]).wait()


def _tc_finish(p_ref, d_ref, x_ref, w_ref, b_ref, o_ref):
    agg = p_ref[0] + p_ref[1]
    deg16 = d_ref[0] + d_ref[1]
    deg = jnp.maximum(deg16[:, 0:1], 1.0)
    h = agg / deg + x_ref[...]
    y = jnp.dot(h, w_ref[...], preferred_element_type=jnp.float32) + b_ref[...]
    o_ref[...] = jnp.maximum(y, 0.0)


def kernel(x, edge_index, W, b):
    ei = edge_index.astype(jnp.int32).reshape(2, NW, NCHUNK, C)
    ones16 = jnp.ones((C, DG), jnp.float32)
    partials, degs = _sc_aggregate(x, ei, ones16)

    BN = 2000
    out = pl.pallas_call(
        _tc_finish,
        grid=(N // BN,),
        in_specs=[
            pl.BlockSpec((NC, BN, D), lambda i: (0, i, 0)),
            pl.BlockSpec((NC, BN, DG), lambda i: (0, i, 0)),
            pl.BlockSpec((BN, D), lambda i: (i, 0)),
            pl.BlockSpec((D, D), lambda i: (0, 0)),
            pl.BlockSpec((1, D), lambda i: (0, 0)),
        ],
        out_specs=pl.BlockSpec((BN, D), lambda i: (i, 0)),
        out_shape=jax.ShapeDtypeStruct((N, D), jnp.float32),
    )(partials, degs, x, W, b.reshape(1, D))
    return out
